# Initial kernel scaffold; baseline (speedup 1.0000x reference)
import functools

import jax
import jax.numpy as jnp
from jax import lax
from jax.experimental import pallas as pl
from jax.experimental.pallas import tpu as pltpu
from jax.experimental.pallas import tpu_sc as plsc

NC = 2
NS = 16
NW = NC * NS

ZROWS = 1000


def _vmesh():
  return plsc.VectorSubcoreMesh(
      core_axis_name="c", subcore_axis_name="s", num_cores=NC, num_subcores=NS)


def _sc_degrees(src, dst, ones, zeros, n):
  e = src.shape[0]
  per_w = e // NW
  k = ones.shape[0]
  steps = per_w // k
  nz = n // ZROWS

  out_sds = jax.ShapeDtypeStruct((NC * n, 16), jnp.float32)

  @functools.partial(
      pl.kernel,
      out_type=(out_sds, out_sds),
      mesh=_vmesh(),
      scratch_types=[
          pltpu.VMEM((k,), jnp.int32),
          pltpu.VMEM((k,), jnp.int32),
          pltpu.VMEM((k, 16), jnp.float32),
          pltpu.VMEM_SHARED((n, 16), jnp.float32),
          pltpu.VMEM_SHARED((n, 16), jnp.float32),
      ],
  )
  def deg_kernel(src_hbm, dst_hbm, ones_hbm, zeros_hbm,
                 dego_hbm, degi_hbm,
                 idx_s, idx_d, ones_v, dego_sh, degi_sh):
    cid = lax.axis_index("c")
    sid = lax.axis_index("s")
    wid = sid * NC + cid
    pltpu.sync_copy(ones_hbm, ones_v)

    @pl.when(sid < nz)
    def _():
      row0 = sid * ZROWS
      pltpu.sync_copy(zeros_hbm, dego_sh.at[pl.ds(row0, ZROWS)])
      pltpu.sync_copy(zeros_hbm, degi_sh.at[pl.ds(row0, ZROWS)])

    plsc.subcore_barrier()

    @pl.loop(0, steps)
    def _(s):
      base = wid * per_w + s * k
      pltpu.sync_copy(src_hbm.at[pl.ds(base, k)], idx_s)
      pltpu.sync_copy(dst_hbm.at[pl.ds(base, k)], idx_d)
      pltpu.sync_copy(ones_v, dego_sh.at[idx_s], add=True)
      pltpu.sync_copy(ones_v, degi_sh.at[idx_d], add=True)

    plsc.subcore_barrier()

    @pl.when(sid < nz)
    def _():
      row0 = sid * ZROWS
      out0 = cid * n + row0
      pltpu.sync_copy(dego_sh.at[pl.ds(row0, ZROWS)],
                      dego_hbm.at[pl.ds(out0, ZROWS)])
      pltpu.sync_copy(degi_sh.at[pl.ds(row0, ZROWS)],
                      degi_hbm.at[pl.ds(out0, ZROWS)])

  return deg_kernel(src, dst, ones, zeros)


def _sc_edge_agg(h, src, dst, zeros, k):
  n, d = h.shape
  e = src.shape[0]
  per_w = e // NW
  steps = per_w // k
  nz = n // ZROWS

  @functools.partial(
      pl.kernel,
      out_type=jax.ShapeDtypeStruct((NC * n, d), jnp.float32),
      mesh=_vmesh(),
      scratch_types=[
          pltpu.VMEM((k,), jnp.int32),
          pltpu.VMEM((k,), jnp.int32),
          pltpu.VMEM((k, d), jnp.float32),
          pltpu.VMEM_SHARED((n, d), jnp.float32),
          pltpu.SemaphoreType.DMA,
      ],
  )
  def agg_kernel(h_hbm, src_hbm, dst_hbm, zeros_hbm, out_hbm,
                 idx_s, idx_d, rows_v, agg_sh, sem):
    cid = lax.axis_index("c")
    sid = lax.axis_index("s")
    wid = sid * NC + cid

    @pl.when(sid < nz)
    def _():
      pltpu.sync_copy(zeros_hbm, agg_sh.at[pl.ds(sid * ZROWS, ZROWS)])

    plsc.subcore_barrier()

    @pl.loop(0, steps)
    def _(s):
      base = wid * per_w + s * k
      pltpu.sync_copy(src_hbm.at[pl.ds(base, k)], idx_s)
      pltpu.sync_copy(dst_hbm.at[pl.ds(base, k)], idx_d)
      pltpu.async_copy(h_hbm.at[idx_s], rows_v, sem).wait()
      pltpu.sync_copy(rows_v, agg_sh.at[idx_d], add=True)

    plsc.subcore_barrier()

    @pl.when(sid < nz)
    def _():
      row0 = sid * ZROWS
      pltpu.sync_copy(agg_sh.at[pl.ds(row0, ZROWS)],
                      out_hbm.at[pl.ds(cid * n + row0, ZROWS)])

  return agg_kernel(h, src, dst, zeros)


def _deg_to_norm(dp, n):
  deg = (dp[:n] + dp[n:]).sum(axis=-1) * 0.0625
  return jnp.where(deg > 0.0, lax.rsqrt(deg), 0.0)


def _tc_matmul(x, w):
  def body(x_ref, w_ref, o_ref):
    o_ref[...] = jnp.dot(x_ref[...], w_ref[...],
                         preferred_element_type=jnp.float32)
  return pl.pallas_call(
      body,
      out_shape=jax.ShapeDtypeStruct((x.shape[0], w.shape[1]), jnp.float32),
  )(x, w)


def _tc_scale_src(xw, dego_p):
  n = xw.shape[0]

  def body(xw_ref, dp_ref, o_ref):
    ns = _deg_to_norm(dp_ref[...], n)
    o_ref[...] = xw_ref[...] * ns[:, None]

  return pl.pallas_call(
      body,
      out_shape=jax.ShapeDtypeStruct(xw.shape, jnp.float32),
  )(xw, dego_p)


def _tc_mid(p1, degi_p, dego_p, b1, w2):
  n = p1.shape[0] // 2
  do = w2.shape[1]

  def body(p_ref, di_ref, dd_ref, b_ref, w_ref, o_ref):
    p = p_ref[...]
    agg = p[:n] + p[n:]
    nd = _deg_to_norm(di_ref[...], n)
    ns = _deg_to_norm(dd_ref[...], n)
    h = jnp.maximum(agg * nd[:, None] + b_ref[...], 0.0)
    o_ref[...] = jnp.dot(h, w_ref[...],
                         preferred_element_type=jnp.float32) * ns[:, None]

  return pl.pallas_call(
      body,
      out_shape=jax.ShapeDtypeStruct((n, do), jnp.float32),
  )(p1, degi_p, dego_p, b1.reshape(1, -1), w2)


def _tc_final(p2, degi_p, b2):
  n = p2.shape[0] // 2
  d = p2.shape[1]

  def body(p_ref, di_ref, b_ref, o_ref):
    p = p_ref[...]
    agg = p[:n] + p[n:]
    nd = _deg_to_norm(di_ref[...], n)
    o_ref[...] = agg * nd[:, None] + b_ref[...]

  return pl.pallas_call(
      body,
      out_shape=jax.ShapeDtypeStruct((n, d), jnp.float32),
  )(p2, degi_p, b2.reshape(1, -1))


def kernel(features, edge_index, W1, b1, W2, b2):
  n = features.shape[0]
  src = edge_index[0].astype(jnp.int32)
  dst = edge_index[1].astype(jnp.int32)

  ones = jnp.ones((1000, 16), jnp.float32)
  z16 = jnp.zeros((ZROWS, 16), jnp.float32)
  zh = jnp.zeros((ZROWS, W1.shape[1]), jnp.float32)
  zo = jnp.zeros((ZROWS, W2.shape[1]), jnp.float32)

  dego_p, degi_p = _sc_degrees(src, dst, ones, z16, n)
  xw1 = _tc_matmul(features, W1)
  h1 = _tc_scale_src(xw1, dego_p)
  p1 = _sc_edge_agg(h1, src, dst, zh, 400)
  h2 = _tc_mid(p1, degi_p, dego_p, b1, W2)
  p2 = _sc_edge_agg(h2, src, dst, zo, 400)
  return _tc_final(p2, degi_p, b2)


# speedup vs baseline: 213.6513x; 213.6513x over previous
import functools

import jax
import jax.numpy as jnp
from jax import lax
from jax.experimental import pallas as pl
from jax.experimental.pallas import tpu as pltpu
from jax.experimental.pallas import tpu_sc as plsc

NC = 2
NS = 16
NW = NC * NS

ZROWS = 1000


def _vmesh():
  return plsc.VectorSubcoreMesh(
      core_axis_name="c", subcore_axis_name="s", num_cores=NC, num_subcores=NS)


def _sc_degrees(src, dst, zeros_n, n):
  e = src.shape[0]
  per_w = e // NW

  out_sds = jax.ShapeDtypeStruct((NW, n), jnp.float32)

  @functools.partial(
      pl.kernel,
      out_type=(out_sds, out_sds),
      mesh=_vmesh(),
      compiler_params=pltpu.CompilerParams(needs_layout_passes=False),
      scratch_types=[
          pltpu.VMEM((per_w,), jnp.int32),
          pltpu.VMEM((per_w,), jnp.int32),
          pltpu.VMEM((n,), jnp.float32),
          pltpu.VMEM((n,), jnp.float32),
      ],
  )
  def deg_kernel(src_hbm, dst_hbm, zeros_hbm,
                 dego_hbm, degi_hbm,
                 idx_s, idx_d, hist_o, hist_i):
    cid = lax.axis_index("c")
    sid = lax.axis_index("s")
    wid = sid * jnp.int32(NC) + cid
    base = wid * jnp.int32(per_w)
    pltpu.sync_copy(src_hbm.at[pl.ds(base, per_w)], idx_s)
    pltpu.sync_copy(dst_hbm.at[pl.ds(base, per_w)], idx_d)
    pltpu.sync_copy(zeros_hbm, hist_o)
    pltpu.sync_copy(zeros_hbm, hist_i)
    ones16 = jnp.full((16,), 1.0, jnp.float32)

    def step_body(i, carry):
      off = i * jnp.int32(16)
      plsc.addupdate_scatter(hist_o, [idx_s[pl.ds(off, 16)]], ones16)
      plsc.addupdate_scatter(hist_i, [idx_d[pl.ds(off, 16)]], ones16)
      return carry

    lax.fori_loop(jnp.int32(0), jnp.int32(per_w // 16), step_body,
                  jnp.int32(0))

    pltpu.sync_copy(hist_o, dego_hbm.at[wid])
    pltpu.sync_copy(hist_i, degi_hbm.at[wid])

  return deg_kernel(src, dst, zeros_n)


def _sc_edge_agg(h, src, dst, zeros, k):
  n, d = h.shape
  e = src.shape[0]
  per_w = e // NW
  steps = per_w // k
  nz = n // ZROWS

  @functools.partial(
      pl.kernel,
      out_type=jax.ShapeDtypeStruct((NC * n, d), jnp.float32),
      mesh=_vmesh(),
      compiler_params=pltpu.CompilerParams(use_tc_tiling_on_sc=False),
      scratch_types=[
          pltpu.VMEM((k,), jnp.int32),
          pltpu.VMEM((k,), jnp.int32),
          pltpu.VMEM((k, d), jnp.float32),
          pltpu.VMEM_SHARED((n, d), jnp.float32),
          pltpu.SemaphoreType.DMA,
      ],
  )
  def agg_kernel(h_hbm, src_hbm, dst_hbm, zeros_hbm, out_hbm,
                 idx_s, idx_d, rows_v, agg_sh, sem):
    cid = lax.axis_index("c")
    sid = lax.axis_index("s")
    wid = sid * jnp.int32(NC) + cid

    @pl.when(sid < jnp.int32(nz))
    def _():
      pltpu.sync_copy(zeros_hbm, agg_sh.at[pl.ds(sid * jnp.int32(ZROWS), ZROWS)])

    plsc.subcore_barrier()

    def step_body(s, carry):
      base = wid * jnp.int32(per_w) + s * jnp.int32(k)
      pltpu.sync_copy(src_hbm.at[pl.ds(base, k)], idx_s)
      pltpu.sync_copy(dst_hbm.at[pl.ds(base, k)], idx_d)
      pltpu.async_copy(h_hbm.at[idx_s], rows_v, sem).wait()
      pltpu.sync_copy(rows_v, agg_sh.at[idx_d], add=True)
      return carry

    lax.fori_loop(jnp.int32(0), jnp.int32(steps), step_body, jnp.int32(0))

    plsc.subcore_barrier()

    @pl.when(sid < jnp.int32(nz))
    def _():
      row0 = sid * jnp.int32(ZROWS)
      pltpu.sync_copy(agg_sh.at[pl.ds(row0, ZROWS)],
                      out_hbm.at[pl.ds(cid * jnp.int32(n) + row0, ZROWS)])

  return agg_kernel(h, src, dst, zeros)


def _deg_to_norm(dp, n):
  del n
  deg = dp.sum(axis=0)
  return jnp.where(deg > 0.0, lax.rsqrt(deg), 0.0)


def _tc_matmul(x, w):
  def body(x_ref, w_ref, o_ref):
    o_ref[...] = jnp.dot(x_ref[...], w_ref[...],
                         preferred_element_type=jnp.float32)
  return pl.pallas_call(
      body,
      out_shape=jax.ShapeDtypeStruct((x.shape[0], w.shape[1]), jnp.float32),
  )(x, w)


def _tc_scale_src(xw, dego_p):
  n = xw.shape[0]

  def body(xw_ref, dp_ref, o_ref):
    ns = _deg_to_norm(dp_ref[...], n)
    o_ref[...] = xw_ref[...] * ns[:, None]

  return pl.pallas_call(
      body,
      out_shape=jax.ShapeDtypeStruct(xw.shape, jnp.float32),
  )(xw, dego_p)


def _tc_mid(p1, degi_p, dego_p, b1, w2):
  n = p1.shape[0] // 2
  do = w2.shape[1]

  def body(p_ref, di_ref, dd_ref, b_ref, w_ref, o_ref):
    p = p_ref[...]
    agg = p[:n] + p[n:]
    nd = _deg_to_norm(di_ref[...], n)
    ns = _deg_to_norm(dd_ref[...], n)
    h = jnp.maximum(agg * nd[:, None] + b_ref[...], 0.0)
    o_ref[...] = jnp.dot(h, w_ref[...],
                         preferred_element_type=jnp.float32) * ns[:, None]

  return pl.pallas_call(
      body,
      out_shape=jax.ShapeDtypeStruct((n, do), jnp.float32),
  )(p1, degi_p, dego_p, b1.reshape(1, -1), w2)


def _tc_final(p2, degi_p, b2):
  n = p2.shape[0] // 2
  d = p2.shape[1]

  def body(p_ref, di_ref, b_ref, o_ref):
    p = p_ref[...]
    agg = p[:n] + p[n:]
    nd = _deg_to_norm(di_ref[...], n)
    o_ref[...] = agg * nd[:, None] + b_ref[...]

  return pl.pallas_call(
      body,
      out_shape=jax.ShapeDtypeStruct((n, d), jnp.float32),
  )(p2, degi_p, b2.reshape(1, -1))


def kernel(features, edge_index, W1, b1, W2, b2):
  n = features.shape[0]
  features = features.astype(jnp.float32)
  W1 = W1.astype(jnp.float32)
  b1 = b1.astype(jnp.float32)
  W2 = W2.astype(jnp.float32)
  b2 = b2.astype(jnp.float32)
  src = edge_index[0].astype(jnp.int32)
  dst = edge_index[1].astype(jnp.int32)

  zn = jnp.zeros((n,), jnp.float32)
  zh = jnp.zeros((ZROWS, W1.shape[1]), jnp.float32)
  zo = jnp.zeros((ZROWS, W2.shape[1]), jnp.float32)

  dego_p, degi_p = _sc_degrees(src, dst, zn, n)
  xw1 = _tc_matmul(features, W1)
  h1 = _tc_scale_src(xw1, dego_p)
  p1 = _sc_edge_agg(h1, src, dst, zh, 200)
  h2 = _tc_mid(p1, degi_p, dego_p, b1, W2)
  p2 = _sc_edge_agg(h2, src, dst, zo, 200)
  return _tc_final(p2, degi_p, b2)


# speedup vs baseline: 253.7008x; 1.1875x over previous
import functools

import jax
import jax.numpy as jnp
from jax import lax
from jax.experimental import pallas as pl
from jax.experimental.pallas import tpu as pltpu
from jax.experimental.pallas import tpu_sc as plsc

NC = 2
NS = 16
NW = NC * NS

ZROWS = 1000


def _vmesh():
  return plsc.VectorSubcoreMesh(
      core_axis_name="c", subcore_axis_name="s", num_cores=NC, num_subcores=NS)


def _sc_degrees(src, dst, zeros_n, n):
  e = src.shape[0]
  per_w = e // NW

  out_sds = jax.ShapeDtypeStruct((NW, n), jnp.float32)

  @functools.partial(
      pl.kernel,
      out_type=(out_sds, out_sds),
      mesh=_vmesh(),
      compiler_params=pltpu.CompilerParams(needs_layout_passes=False),
      scratch_types=[
          pltpu.VMEM((per_w,), jnp.int32),
          pltpu.VMEM((per_w,), jnp.int32),
          pltpu.VMEM((n,), jnp.float32),
          pltpu.VMEM((n,), jnp.float32),
      ],
  )
  def deg_kernel(src_hbm, dst_hbm, zeros_hbm,
                 dego_hbm, degi_hbm,
                 idx_s, idx_d, hist_o, hist_i):
    cid = lax.axis_index("c")
    sid = lax.axis_index("s")
    wid = sid * jnp.int32(NC) + cid
    base = wid * jnp.int32(per_w)
    pltpu.sync_copy(src_hbm.at[pl.ds(base, per_w)], idx_s)
    pltpu.sync_copy(dst_hbm.at[pl.ds(base, per_w)], idx_d)
    pltpu.sync_copy(zeros_hbm, hist_o)
    pltpu.sync_copy(zeros_hbm, hist_i)
    ones16 = jnp.full((16,), 1.0, jnp.float32)

    def step_body(i, carry):
      off = i * jnp.int32(16)
      plsc.addupdate_scatter(hist_o, [idx_s[pl.ds(off, 16)]], ones16)
      plsc.addupdate_scatter(hist_i, [idx_d[pl.ds(off, 16)]], ones16)
      return carry

    lax.fori_loop(jnp.int32(0), jnp.int32(per_w // 16), step_body,
                  jnp.int32(0))

    pltpu.sync_copy(hist_o, dego_hbm.at[wid])
    pltpu.sync_copy(hist_i, degi_hbm.at[wid])

  return deg_kernel(src, dst, zeros_n)


def _sc_edge_agg(h, src, dst, zeros, k):
  n, d = h.shape
  e = src.shape[0]
  per_w = e // NW
  steps = per_w // k
  nz = n // ZROWS

  pairs = steps // 2

  @functools.partial(
      pl.kernel,
      out_type=jax.ShapeDtypeStruct((NC * n, d), jnp.float32),
      mesh=_vmesh(),
      compiler_params=pltpu.CompilerParams(use_tc_tiling_on_sc=False),
      scratch_types=[
          pltpu.VMEM((k,), jnp.int32),
          pltpu.VMEM((k,), jnp.int32),
          pltpu.VMEM((k,), jnp.int32),
          pltpu.VMEM((k,), jnp.int32),
          pltpu.VMEM((k, d), jnp.float32),
          pltpu.VMEM((k, d), jnp.float32),
          pltpu.VMEM_SHARED((n, d), jnp.float32),
          pltpu.SemaphoreType.DMA,
          pltpu.SemaphoreType.DMA,
      ],
  )
  def agg_kernel(h_hbm, src_hbm, dst_hbm, zeros_hbm, out_hbm,
                 idx_s0, idx_d0, idx_s1, idx_d1, rows0, rows1,
                 agg_sh, sem0, sem1):
    cid = lax.axis_index("c")
    sid = lax.axis_index("s")
    wid = sid * jnp.int32(NC) + cid
    base0 = wid * jnp.int32(per_w)

    @pl.when(sid < jnp.int32(nz))
    def _():
      pltpu.sync_copy(zeros_hbm, agg_sh.at[pl.ds(sid * jnp.int32(ZROWS), ZROWS)])

    plsc.subcore_barrier()

    pltpu.sync_copy(src_hbm.at[pl.ds(base0, k)], idx_s0)
    pltpu.sync_copy(dst_hbm.at[pl.ds(base0, k)], idx_d0)
    pltpu.async_copy(h_hbm.at[idx_s0], rows0, sem0)

    def pair_body(i, carry):
      b1 = base0 + (jnp.int32(2) * i + jnp.int32(1)) * jnp.int32(k)
      pltpu.sync_copy(src_hbm.at[pl.ds(b1, k)], idx_s1)
      pltpu.sync_copy(dst_hbm.at[pl.ds(b1, k)], idx_d1)
      pltpu.async_copy(h_hbm.at[idx_s1], rows1, sem1)
      pltpu.make_async_copy(h_hbm.at[idx_s0], rows0, sem0).wait()
      pltpu.sync_copy(rows0, agg_sh.at[idx_d0], add=True)

      @pl.when(i < jnp.int32(pairs - 1))
      def _():
        b2 = base0 + (jnp.int32(2) * i + jnp.int32(2)) * jnp.int32(k)
        pltpu.sync_copy(src_hbm.at[pl.ds(b2, k)], idx_s0)
        pltpu.sync_copy(dst_hbm.at[pl.ds(b2, k)], idx_d0)
        pltpu.async_copy(h_hbm.at[idx_s0], rows0, sem0)

      pltpu.make_async_copy(h_hbm.at[idx_s1], rows1, sem1).wait()
      pltpu.sync_copy(rows1, agg_sh.at[idx_d1], add=True)
      return carry

    lax.fori_loop(jnp.int32(0), jnp.int32(pairs), pair_body, jnp.int32(0))

    if steps % 2 == 1:
      blast = base0 + jnp.int32((steps - 1) * k)
      pltpu.sync_copy(src_hbm.at[pl.ds(blast, k)], idx_s0)
      pltpu.sync_copy(dst_hbm.at[pl.ds(blast, k)], idx_d0)
      pltpu.async_copy(h_hbm.at[idx_s0], rows0, sem0).wait()
      pltpu.sync_copy(rows0, agg_sh.at[idx_d0], add=True)

    plsc.subcore_barrier()

    @pl.when(sid < jnp.int32(nz))
    def _():
      row0 = sid * jnp.int32(ZROWS)
      pltpu.sync_copy(agg_sh.at[pl.ds(row0, ZROWS)],
                      out_hbm.at[pl.ds(cid * jnp.int32(n) + row0, ZROWS)])

  return agg_kernel(h, src, dst, zeros)


def _deg_to_norm(dp, n):
  del n
  deg = dp.sum(axis=0)
  return jnp.where(deg > 0.0, lax.rsqrt(deg), 0.0)


def _tc_matmul(x, w):
  def body(x_ref, w_ref, o_ref):
    o_ref[...] = jnp.dot(x_ref[...], w_ref[...],
                         preferred_element_type=jnp.float32)
  return pl.pallas_call(
      body,
      out_shape=jax.ShapeDtypeStruct((x.shape[0], w.shape[1]), jnp.float32),
  )(x, w)


def _tc_scale_src(xw, dego_p):
  n = xw.shape[0]

  def body(xw_ref, dp_ref, o_ref):
    ns = _deg_to_norm(dp_ref[...], n)
    o_ref[...] = xw_ref[...] * ns[:, None]

  return pl.pallas_call(
      body,
      out_shape=jax.ShapeDtypeStruct(xw.shape, jnp.float32),
  )(xw, dego_p)


def _tc_mid(p1, degi_p, dego_p, b1, w2):
  n = p1.shape[0] // 2
  do = w2.shape[1]

  def body(p_ref, di_ref, dd_ref, b_ref, w_ref, o_ref):
    p = p_ref[...]
    agg = p[:n] + p[n:]
    nd = _deg_to_norm(di_ref[...], n)
    ns = _deg_to_norm(dd_ref[...], n)
    h = jnp.maximum(agg * nd[:, None] + b_ref[...], 0.0)
    o_ref[...] = jnp.dot(h, w_ref[...],
                         preferred_element_type=jnp.float32) * ns[:, None]

  return pl.pallas_call(
      body,
      out_shape=jax.ShapeDtypeStruct((n, do), jnp.float32),
  )(p1, degi_p, dego_p, b1.reshape(1, -1), w2)


def _tc_final(p2, degi_p, b2):
  n = p2.shape[0] // 2
  d = p2.shape[1]

  def body(p_ref, di_ref, b_ref, o_ref):
    p = p_ref[...]
    agg = p[:n] + p[n:]
    nd = _deg_to_norm(di_ref[...], n)
    o_ref[...] = agg * nd[:, None] + b_ref[...]

  return pl.pallas_call(
      body,
      out_shape=jax.ShapeDtypeStruct((n, d), jnp.float32),
  )(p2, degi_p, b2.reshape(1, -1))


def kernel(features, edge_index, W1, b1, W2, b2):
  n = features.shape[0]
  features = features.astype(jnp.float32)
  W1 = W1.astype(jnp.float32)
  b1 = b1.astype(jnp.float32)
  W2 = W2.astype(jnp.float32)
  b2 = b2.astype(jnp.float32)
  src = edge_index[0].astype(jnp.int32)
  dst = edge_index[1].astype(jnp.int32)

  zn = jnp.zeros((n,), jnp.float32)
  zh = jnp.zeros((ZROWS, W1.shape[1]), jnp.float32)
  zo = jnp.zeros((ZROWS, W2.shape[1]), jnp.float32)

  dego_p, degi_p = _sc_degrees(src, dst, zn, n)
  xw1 = _tc_matmul(features, W1)
  h1 = _tc_scale_src(xw1, dego_p)
  p1 = _sc_edge_agg(h1, src, dst, zh, 80)
  h2 = _tc_mid(p1, degi_p, dego_p, b1, W2)
  p2 = _sc_edge_agg(h2, src, dst, zo, 200)
  return _tc_final(p2, degi_p, b2)


# speedup vs baseline: 299.3808x; 1.1801x over previous
import functools

import jax
import jax.numpy as jnp
from jax import lax
from jax.experimental import pallas as pl
from jax.experimental.pallas import tpu as pltpu
from jax.experimental.pallas import tpu_sc as plsc

NC = 2
NS = 16
NW = NC * NS

ZROWS = 1000


def _vmesh():
  return plsc.VectorSubcoreMesh(
      core_axis_name="c", subcore_axis_name="s", num_cores=NC, num_subcores=NS)


def _sc_degrees(src, dst, zeros_n, n):
  e = src.shape[0]
  per_w = e // NW

  out_sds = jax.ShapeDtypeStruct((NW, n), jnp.float32)

  @functools.partial(
      pl.kernel,
      out_type=(out_sds, out_sds),
      mesh=_vmesh(),
      compiler_params=pltpu.CompilerParams(needs_layout_passes=False),
      scratch_types=[
          pltpu.VMEM((per_w,), jnp.int32),
          pltpu.VMEM((per_w,), jnp.int32),
          pltpu.VMEM((n,), jnp.float32),
          pltpu.VMEM((n,), jnp.float32),
      ],
  )
  def deg_kernel(src_hbm, dst_hbm, zeros_hbm,
                 dego_hbm, degi_hbm,
                 idx_s, idx_d, hist_o, hist_i):
    cid = lax.axis_index("c")
    sid = lax.axis_index("s")
    wid = sid * jnp.int32(NC) + cid
    base = wid * jnp.int32(per_w)
    pltpu.sync_copy(src_hbm.at[pl.ds(base, per_w)], idx_s)
    pltpu.sync_copy(dst_hbm.at[pl.ds(base, per_w)], idx_d)
    pltpu.sync_copy(zeros_hbm, hist_o)
    pltpu.sync_copy(zeros_hbm, hist_i)
    ones16 = jnp.full((16,), 1.0, jnp.float32)

    def step_body(i, carry):
      off = i * jnp.int32(16)
      plsc.addupdate_scatter(hist_o, [idx_s[pl.ds(off, 16)]], ones16)
      plsc.addupdate_scatter(hist_i, [idx_d[pl.ds(off, 16)]], ones16)
      return carry

    lax.fori_loop(jnp.int32(0), jnp.int32(per_w // 16), step_body,
                  jnp.int32(0))

    pltpu.sync_copy(hist_o, dego_hbm.at[wid])
    pltpu.sync_copy(hist_i, degi_hbm.at[wid])

  return deg_kernel(src, dst, zeros_n)


def _sc_edge_agg(h, src, dst, zeros, k):
  n, d = h.shape
  e = src.shape[0]
  per_w = e // NW
  steps = per_w // k
  rem = per_w % k
  nz = n // ZROWS

  pairs = steps // 2

  scratch = [
      pltpu.VMEM((k,), jnp.int32),
      pltpu.VMEM((k,), jnp.int32),
      pltpu.VMEM((k,), jnp.int32),
      pltpu.VMEM((k,), jnp.int32),
      pltpu.VMEM((k, d), jnp.float32),
      pltpu.VMEM((k, d), jnp.float32),
      pltpu.VMEM_SHARED((n, d), jnp.float32),
      pltpu.SemaphoreType.DMA,
      pltpu.SemaphoreType.DMA,
  ]
  if rem:
    scratch.append(pltpu.VMEM((rem,), jnp.int32))

  @functools.partial(
      pl.kernel,
      out_type=jax.ShapeDtypeStruct((NC * n, d), jnp.float32),
      mesh=_vmesh(),
      compiler_params=pltpu.CompilerParams(use_tc_tiling_on_sc=False),
      scratch_types=scratch,
  )
  def agg_kernel(h_hbm, src_hbm, dst_hbm, zeros_hbm, out_hbm,
                 idx_s0, idx_d0, idx_s1, idx_d1, rows0, rows1,
                 agg_sh, sem0, sem1, *rest):
    cid = lax.axis_index("c")
    sid = lax.axis_index("s")
    wid = sid * jnp.int32(NC) + cid
    base0 = wid * jnp.int32(per_w)

    @pl.when(sid < jnp.int32(nz))
    def _():
      pltpu.sync_copy(zeros_hbm, agg_sh.at[pl.ds(sid * jnp.int32(ZROWS), ZROWS)])

    plsc.subcore_barrier()

    pltpu.sync_copy(src_hbm.at[pl.ds(base0, k)], idx_s0)
    pltpu.sync_copy(dst_hbm.at[pl.ds(base0, k)], idx_d0)
    pltpu.async_copy(h_hbm.at[idx_s0], rows0, sem0)

    def pair_body(i, carry):
      b1 = base0 + (jnp.int32(2) * i + jnp.int32(1)) * jnp.int32(k)
      pltpu.sync_copy(src_hbm.at[pl.ds(b1, k)], idx_s1)
      pltpu.sync_copy(dst_hbm.at[pl.ds(b1, k)], idx_d1)
      pltpu.async_copy(h_hbm.at[idx_s1], rows1, sem1)
      pltpu.make_async_copy(h_hbm.at[idx_s0], rows0, sem0).wait()
      pltpu.sync_copy(rows0, agg_sh.at[idx_d0], add=True)

      @pl.when(i < jnp.int32(pairs - 1))
      def _():
        b2 = base0 + (jnp.int32(2) * i + jnp.int32(2)) * jnp.int32(k)
        pltpu.sync_copy(src_hbm.at[pl.ds(b2, k)], idx_s0)
        pltpu.sync_copy(dst_hbm.at[pl.ds(b2, k)], idx_d0)
        pltpu.async_copy(h_hbm.at[idx_s0], rows0, sem0)

      pltpu.make_async_copy(h_hbm.at[idx_s1], rows1, sem1).wait()
      pltpu.sync_copy(rows1, agg_sh.at[idx_d1], add=True)
      return carry

    lax.fori_loop(jnp.int32(0), jnp.int32(pairs), pair_body, jnp.int32(0))

    if steps % 2 == 1:
      blast = base0 + jnp.int32((steps - 1) * k)
      pltpu.sync_copy(src_hbm.at[pl.ds(blast, k)], idx_s0)
      pltpu.sync_copy(dst_hbm.at[pl.ds(blast, k)], idx_d0)
      pltpu.async_copy(h_hbm.at[idx_s0], rows0, sem0).wait()
      pltpu.sync_copy(rows0, agg_sh.at[idx_d0], add=True)

    if rem:
      idx_de = rest[0]
      bt = base0 + jnp.int32(per_w - k)
      pltpu.sync_copy(src_hbm.at[pl.ds(bt, k)], idx_s0)
      pltpu.sync_copy(dst_hbm.at[pl.ds(bt + jnp.int32(k - rem), rem)], idx_de)
      pltpu.async_copy(h_hbm.at[idx_s0], rows0, sem0).wait()
      pltpu.sync_copy(rows0.at[pl.ds(k - rem, rem)], agg_sh.at[idx_de],
                      add=True)

    plsc.subcore_barrier()

    @pl.when(sid < jnp.int32(nz))
    def _():
      row0 = sid * jnp.int32(ZROWS)
      pltpu.sync_copy(agg_sh.at[pl.ds(row0, ZROWS)],
                      out_hbm.at[pl.ds(cid * jnp.int32(n) + row0, ZROWS)])

  return agg_kernel(h, src, dst, zeros)


def _deg_to_norm(dp, n):
  del n
  deg = dp.sum(axis=0)
  return jnp.where(deg > 0.0, lax.rsqrt(deg), 0.0)


def _tc_layer1(x, w, dego_p):
  n = x.shape[0]

  def body(x_ref, w_ref, dp_ref, o_ref):
    ns = _deg_to_norm(dp_ref[...], n)
    o_ref[...] = jnp.dot(x_ref[...] * ns[:, None], w_ref[...],
                         preferred_element_type=jnp.float32)

  return pl.pallas_call(
      body,
      out_shape=jax.ShapeDtypeStruct((n, w.shape[1]), jnp.float32),
  )(x, w, dego_p)


def _tc_mid(p1, degi_p, dego_p, b1, w2):
  n = p1.shape[0] // 2
  do = w2.shape[1]

  def body(p_ref, di_ref, dd_ref, b_ref, w_ref, o_ref):
    p = p_ref[...]
    agg = p[:n] + p[n:]
    nd = _deg_to_norm(di_ref[...], n)
    ns = _deg_to_norm(dd_ref[...], n)
    h = jnp.maximum(agg * nd[:, None] + b_ref[...], 0.0)
    o_ref[...] = jnp.dot(h, w_ref[...],
                         preferred_element_type=jnp.float32) * ns[:, None]

  return pl.pallas_call(
      body,
      out_shape=jax.ShapeDtypeStruct((n, do), jnp.float32),
  )(p1, degi_p, dego_p, b1.reshape(1, -1), w2)


def _tc_final(p2, degi_p, b2):
  n = p2.shape[0] // 2
  d = p2.shape[1]

  def body(p_ref, di_ref, b_ref, o_ref):
    p = p_ref[...]
    agg = p[:n] + p[n:]
    nd = _deg_to_norm(di_ref[...], n)
    o_ref[...] = agg * nd[:, None] + b_ref[...]

  return pl.pallas_call(
      body,
      out_shape=jax.ShapeDtypeStruct((n, d), jnp.float32),
  )(p2, degi_p, b2.reshape(1, -1))


def kernel(features, edge_index, W1, b1, W2, b2):
  n = features.shape[0]
  features = features.astype(jnp.float32)
  W1 = W1.astype(jnp.float32)
  b1 = b1.astype(jnp.float32)
  W2 = W2.astype(jnp.float32)
  b2 = b2.astype(jnp.float32)
  src = edge_index[0].astype(jnp.int32)
  dst = edge_index[1].astype(jnp.int32)

  zn = jnp.zeros((n,), jnp.float32)
  zh = jnp.zeros((ZROWS, W1.shape[1]), jnp.float32)
  zo = jnp.zeros((ZROWS, W2.shape[1]), jnp.float32)

  dego_p, degi_p = _sc_degrees(src, dst, zn, n)
  h1 = _tc_layer1(features, W1, dego_p)
  p1 = _sc_edge_agg(h1, src, dst, zh, 192)
  h2 = _tc_mid(p1, degi_p, dego_p, b1, W2)
  p2 = _sc_edge_agg(h2, src, dst, zo, 200)
  return _tc_final(p2, degi_p, b2)


# speedup vs baseline: 311.9475x; 1.0420x over previous
import functools

import jax
import jax.numpy as jnp
from jax import lax
from jax.experimental import pallas as pl
from jax.experimental.pallas import tpu as pltpu
from jax.experimental.pallas import tpu_sc as plsc

NC = 2
NS = 16
NW = NC * NS

ZROWS = 1000


def _vmesh():
  return plsc.VectorSubcoreMesh(
      core_axis_name="c", subcore_axis_name="s", num_cores=NC, num_subcores=NS)


def _sc_degrees(src, dst, zeros_n, n):
  e = src.shape[0]
  per_w = e // NW

  out_sds = jax.ShapeDtypeStruct((NW, n), jnp.float32)

  @functools.partial(
      pl.kernel,
      out_type=(out_sds, out_sds),
      mesh=_vmesh(),
      compiler_params=pltpu.CompilerParams(needs_layout_passes=False),
      scratch_types=[
          pltpu.VMEM((per_w,), jnp.int32),
          pltpu.VMEM((per_w,), jnp.int32),
          pltpu.VMEM((n,), jnp.float32),
          pltpu.VMEM((n,), jnp.float32),
      ],
  )
  def deg_kernel(src_hbm, dst_hbm, zeros_hbm,
                 dego_hbm, degi_hbm,
                 idx_s, idx_d, hist_o, hist_i):
    cid = lax.axis_index("c")
    sid = lax.axis_index("s")
    wid = sid * jnp.int32(NC) + cid
    base = wid * jnp.int32(per_w)
    pltpu.sync_copy(src_hbm.at[pl.ds(base, per_w)], idx_s)
    pltpu.sync_copy(dst_hbm.at[pl.ds(base, per_w)], idx_d)
    pltpu.sync_copy(zeros_hbm, hist_o)
    pltpu.sync_copy(zeros_hbm, hist_i)
    ones16 = jnp.full((16,), 1.0, jnp.float32)

    def step_body(i, carry):
      off = i * jnp.int32(16)
      plsc.addupdate_scatter(hist_o, [idx_s[pl.ds(off, 16)]], ones16)
      plsc.addupdate_scatter(hist_i, [idx_d[pl.ds(off, 16)]], ones16)
      return carry

    lax.fori_loop(jnp.int32(0), jnp.int32(per_w // 16), step_body,
                  jnp.int32(0))

    pltpu.sync_copy(hist_o, dego_hbm.at[wid])
    pltpu.sync_copy(hist_i, degi_hbm.at[wid])

  return deg_kernel(src, dst, zeros_n)


def _sc_edge_agg(h, src, dst, zeros, k):
  n, d = h.shape
  e = src.shape[0]
  per_w = e // NW
  steps = per_w // k
  rem = per_w % k
  nz = n // ZROWS

  pairs = steps // 2

  scratch = [
      pltpu.VMEM((k,), jnp.int32),
      pltpu.VMEM((k,), jnp.int32),
      pltpu.VMEM((k,), jnp.int32),
      pltpu.VMEM((k,), jnp.int32),
      pltpu.VMEM((k, d), jnp.float32),
      pltpu.VMEM((k, d), jnp.float32),
      pltpu.VMEM_SHARED((n, d), jnp.float32),
      pltpu.SemaphoreType.DMA,
      pltpu.SemaphoreType.DMA,
  ]
  if rem:
    scratch.append(pltpu.VMEM((rem,), jnp.int32))

  @functools.partial(
      pl.kernel,
      out_type=jax.ShapeDtypeStruct((NC * n, d), jnp.float32),
      mesh=_vmesh(),
      compiler_params=pltpu.CompilerParams(use_tc_tiling_on_sc=False),
      scratch_types=scratch,
  )
  def agg_kernel(h_hbm, src_hbm, dst_hbm, zeros_hbm, out_hbm,
                 idx_s0, idx_d0, idx_s1, idx_d1, rows0, rows1,
                 agg_sh, sem0, sem1, *rest):
    cid = lax.axis_index("c")
    sid = lax.axis_index("s")
    wid = sid * jnp.int32(NC) + cid
    base0 = wid * jnp.int32(per_w)

    @pl.when(sid < jnp.int32(nz))
    def _():
      pltpu.sync_copy(zeros_hbm, agg_sh.at[pl.ds(sid * jnp.int32(ZROWS), ZROWS)])

    plsc.subcore_barrier()

    pltpu.sync_copy(src_hbm.at[pl.ds(base0, k)], idx_s0)
    pltpu.sync_copy(dst_hbm.at[pl.ds(base0, k)], idx_d0)
    pltpu.async_copy(h_hbm.at[idx_s0], rows0, sem0)

    def pair_body(i, carry):
      b1 = base0 + (jnp.int32(2) * i + jnp.int32(1)) * jnp.int32(k)
      pltpu.sync_copy(src_hbm.at[pl.ds(b1, k)], idx_s1)
      pltpu.sync_copy(dst_hbm.at[pl.ds(b1, k)], idx_d1)
      pltpu.async_copy(h_hbm.at[idx_s1], rows1, sem1)
      pltpu.make_async_copy(h_hbm.at[idx_s0], rows0, sem0).wait()
      pltpu.sync_copy(rows0, agg_sh.at[idx_d0], add=True)

      @pl.when(i < jnp.int32(pairs - 1))
      def _():
        b2 = base0 + (jnp.int32(2) * i + jnp.int32(2)) * jnp.int32(k)
        pltpu.sync_copy(src_hbm.at[pl.ds(b2, k)], idx_s0)
        pltpu.sync_copy(dst_hbm.at[pl.ds(b2, k)], idx_d0)
        pltpu.async_copy(h_hbm.at[idx_s0], rows0, sem0)

      pltpu.make_async_copy(h_hbm.at[idx_s1], rows1, sem1).wait()
      pltpu.sync_copy(rows1, agg_sh.at[idx_d1], add=True)
      return carry

    lax.fori_loop(jnp.int32(0), jnp.int32(pairs), pair_body, jnp.int32(0))

    if steps % 2 == 1:
      blast = base0 + jnp.int32((steps - 1) * k)
      pltpu.sync_copy(src_hbm.at[pl.ds(blast, k)], idx_s0)
      pltpu.sync_copy(dst_hbm.at[pl.ds(blast, k)], idx_d0)
      pltpu.async_copy(h_hbm.at[idx_s0], rows0, sem0).wait()
      pltpu.sync_copy(rows0, agg_sh.at[idx_d0], add=True)

    if rem:
      idx_de = rest[0]
      bt = base0 + jnp.int32(per_w - k)
      pltpu.sync_copy(src_hbm.at[pl.ds(bt, k)], idx_s0)
      pltpu.sync_copy(dst_hbm.at[pl.ds(bt + jnp.int32(k - rem), rem)], idx_de)
      pltpu.async_copy(h_hbm.at[idx_s0], rows0, sem0).wait()
      pltpu.sync_copy(rows0.at[pl.ds(k - rem, rem)], agg_sh.at[idx_de],
                      add=True)

    plsc.subcore_barrier()

    @pl.when(sid < jnp.int32(nz))
    def _():
      row0 = sid * jnp.int32(ZROWS)
      pltpu.sync_copy(agg_sh.at[pl.ds(row0, ZROWS)],
                      out_hbm.at[pl.ds(cid * jnp.int32(n) + row0, ZROWS)])

  return agg_kernel(h, src, dst, zeros)


def _deg_to_norm(dp, n):
  del n
  deg = dp.sum(axis=0)
  return jnp.where(deg > 0.0, lax.rsqrt(deg), 0.0)


def _tc_layer1(x, w, dego_p):
  n = x.shape[0]

  def body(x_ref, w_ref, dp_ref, o_ref):
    ns = _deg_to_norm(dp_ref[...], n)
    o_ref[...] = jnp.dot(x_ref[...] * ns[:, None], w_ref[...],
                         preferred_element_type=jnp.float32)

  return pl.pallas_call(
      body,
      out_shape=jax.ShapeDtypeStruct((n, w.shape[1]), jnp.float32),
  )(x, w, dego_p)


def _tc_mid(p1, degi_p, dego_p, b1, w2):
  n = p1.shape[0] // 2
  do = w2.shape[1]

  def body(p_ref, di_ref, dd_ref, b_ref, w_ref, o_ref):
    p = p_ref[...]
    agg = p[:n] + p[n:]
    nd = _deg_to_norm(di_ref[...], n)
    ns = _deg_to_norm(dd_ref[...], n)
    h = jnp.maximum(agg * nd[:, None] + b_ref[...], 0.0)
    o_ref[...] = jnp.dot(h, w_ref[...],
                         preferred_element_type=jnp.float32) * ns[:, None]

  return pl.pallas_call(
      body,
      out_shape=jax.ShapeDtypeStruct((n, do), jnp.float32),
  )(p1, degi_p, dego_p, b1.reshape(1, -1), w2)


def _tc_final(p2, degi_p, b2):
  n = p2.shape[0] // 2
  d = p2.shape[1]

  def body(p_ref, di_ref, b_ref, o_ref):
    p = p_ref[...]
    agg = p[:n] + p[n:]
    nd = _deg_to_norm(di_ref[...], n)
    o_ref[...] = agg * nd[:, None] + b_ref[...]

  return pl.pallas_call(
      body,
      out_shape=jax.ShapeDtypeStruct((n, d), jnp.float32),
  )(p2, degi_p, b2.reshape(1, -1))


def kernel(features, edge_index, W1, b1, W2, b2):
  n = features.shape[0]
  features = features.astype(jnp.float32)
  W1 = W1.astype(jnp.float32)
  b1 = b1.astype(jnp.float32)
  W2 = W2.astype(jnp.float32)
  b2 = b2.astype(jnp.float32)
  src = edge_index[0].astype(jnp.int32)
  dst = edge_index[1].astype(jnp.int32)

  zn = jnp.zeros((n,), jnp.float32)
  zh = jnp.zeros((ZROWS, W1.shape[1]), jnp.float32)
  zo = jnp.zeros((ZROWS, W2.shape[1]), jnp.float32)

  dego_p, degi_p = _sc_degrees(src, dst, zn, n)
  h1 = _tc_layer1(features, W1, dego_p)
  p1 = _sc_edge_agg(h1, src, dst, zh, 192)
  h2 = _tc_mid(p1, degi_p, dego_p, b1, W2)
  p2 = _sc_edge_agg(h2, src, dst, zo, 344)
  return _tc_final(p2, degi_p, b2)
